# baseline (speedup 1.0000x reference)
import jax
import jax.numpy as jnp
from jax import lax
from jax.experimental import pallas as pl
from jax.experimental.pallas import tpu as pltpu
from jax.experimental.pallas import tpu_sc as plsc

B, H, W, C = 8, 112, 112, 384
HO, WO = H // 2, W // 2
NW = 32
PAIRS = B * HO
PER_W = PAIRS // NW
CO = 14
CI = 2 * CO
KPP = WO // CO
NT = PER_W * KPP
LANES = 16
GRP = C // LANES
UNROLL = 2


def _unpool_body(pool_hbm, rec_hbm, out_hbm,
                 in0, in1, rec0, rec1, o0, o1,
                 si0, si1, sr0, sr1, so0, so1):
    wid = lax.axis_index("s") * 2 + lax.axis_index("c")
    ins, recs, outs = (in0, in1), (rec0, rec1), (o0, o1)
    sis, srs, sos = (si0, si1), (sr0, sr1), (so0, so1)

    def idx(t):
        pair = wid * PER_W + t // KPP
        k = t % KPP
        b = pair // HO
        i = pair - b * HO
        return b, i, k

    def in_copies(t, buf):
        b, i, k = idx(t)
        row = (b * H + 2 * i) * W + CI * k
        return (
            pltpu.make_async_copy(
                pool_hbm.at[pl.ds(row * C, CI * C)], ins[buf].at[0], sis[buf]),
            pltpu.make_async_copy(
                pool_hbm.at[pl.ds((row + W) * C, CI * C)], ins[buf].at[1], sis[buf]),
            pltpu.make_async_copy(
                rec_hbm.at[pl.ds(((b * HO + i) * WO + CO * k) * C, CO * C)],
                recs[buf], srs[buf]),
        )

    def out_copies(t, buf):
        b, i, k = idx(t)
        row = (b * H + 2 * i) * W + CI * k
        return (
            pltpu.make_async_copy(
                outs[buf].at[0], out_hbm.at[pl.ds(row * C, CI * C)], sos[buf]),
            pltpu.make_async_copy(
                outs[buf].at[1], out_hbm.at[pl.ds((row + W) * C, CI * C)], sos[buf]),
        )

    def start_in(t, buf):
        for cp in in_copies(t, buf):
            cp.start()

    def wait_in(t, buf):
        for cp in in_copies(t, buf):
            cp.wait()

    def start_out(t, buf):
        for cp in out_copies(t, buf):
            cp.start()

    def wait_out(t, buf):
        for cp in out_copies(t, buf):
            cp.wait()

    def compute(buf):
        ib, rb, ob = ins[buf], recs[buf], outs[buf]

        @plsc.parallel_loop(0, GRP, step=UNROLL, unroll=2)
        def grp_loop(g0):
            for jw in range(CO):
                for u in range(UNROLL):
                    sl = pl.ds(g0 * LANES + u * LANES, LANES)
                    sle = pl.ds(2 * jw * C + g0 * LANES + u * LANES, LANES)
                    slo = pl.ds((2 * jw + 1) * C + g0 * LANES + u * LANES, LANES)
                    slr = pl.ds(jw * C + g0 * LANES + u * LANES, LANES)
                    a = ib[0, sle]
                    bb = ib[0, slo]
                    cc = ib[1, sle]
                    dd = ib[1, slo]
                    r = rb[slr]
                    m = jnp.maximum(jnp.maximum(a, bb), jnp.maximum(cc, dd))
                    z = jnp.zeros((LANES,), jnp.float32)
                    c0 = a == m
                    c1 = bb == m
                    c2 = cc == m
                    t1 = jnp.where(c0, z, r)
                    t2 = jnp.where(c1, z, t1)
                    ob[0, sle] = jnp.where(c0, r, z)
                    ob[0, slo] = jnp.where(c1, t1, z)
                    ob[1, sle] = jnp.where(c2, t2, z)
                    ob[1, slo] = jnp.where(c2, z, t2)

    start_in(0, 0)

    def main_loop(tt, carry):
        for s in range(2):
            t = 2 * tt + s
            buf = s
            if s == 0:
                start_in(t + 1, 1)
            else:
                @pl.when(tt < NT // 2 - 1)
                def _():
                    start_in(t + 1, 0)

            @pl.when(tt >= 1)
            def _():
                wait_out(t - 2, buf)

            wait_in(t, buf)
            compute(buf)
            start_out(t, buf)
        return carry

    lax.fori_loop(0, NT // 2, main_loop, 0)
    wait_out(NT - 2, 0)
    wait_out(NT - 1, 1)


@jax.jit
def kernel(pool_input, recreated_output):
    run = pl.kernel(
        _unpool_body,
        out_type=jax.ShapeDtypeStruct((B * H * W * C,), jnp.float32),
        scratch_types=[
            pltpu.VMEM((2, CI * C), jnp.float32),
            pltpu.VMEM((2, CI * C), jnp.float32),
            pltpu.VMEM((CO * C,), jnp.float32),
            pltpu.VMEM((CO * C,), jnp.float32),
            pltpu.VMEM((2, CI * C), jnp.float32),
            pltpu.VMEM((2, CI * C), jnp.float32),
            pltpu.SemaphoreType.DMA,
            pltpu.SemaphoreType.DMA,
            pltpu.SemaphoreType.DMA,
            pltpu.SemaphoreType.DMA,
            pltpu.SemaphoreType.DMA,
            pltpu.SemaphoreType.DMA,
        ],
        mesh=plsc.VectorSubcoreMesh(core_axis_name="c", subcore_axis_name="s"),
    )
    out = run(pool_input.reshape(-1), recreated_output.reshape(-1))
    return out.reshape(B, H, W, C)


# speedup vs baseline: 3.2756x; 3.2756x over previous
import jax
import jax.numpy as jnp
from jax import lax
from jax.experimental import pallas as pl
from jax.experimental.pallas import tpu as pltpu
from jax.experimental.pallas import tpu_sc as plsc

B, H, W, C = 8, 112, 112, 384
HO, WO = H // 2, W // 2
NW = 32
PAIRS = B * HO
PER_W = PAIRS // NW
SZ = (16, 16, 16, 8)
OFF = (0, 16, 32, 48)
CMAX = 16
LANES = 16
GRP = C // LANES


def _unpool_body(pool_hbm, rec_hbm, out_hbm,
                 in0, in1, rec0, rec1, o0, o1,
                 si0, si1, sr0, sr1, so0, so1):
    wid = lax.axis_index("s") * 2 + lax.axis_index("c")
    ins, recs, outs = (in0, in1), (rec0, rec1), (o0, o1)
    sis, srs, sos = (si0, si1), (sr0, sr1), (so0, so1)

    def rowcol(p, j):
        pair = wid * PER_W + p
        b = pair // HO
        i = pair - b * HO
        return b, i, OFF[j]

    def in_copies(p, j, buf):
        b, i, co = rowcol(p, j)
        ci = 2 * SZ[j]
        return (
            pltpu.make_async_copy(
                pool_hbm.at[b, pl.ds(2 * i, 2), pl.ds(2 * co, ci)],
                ins[buf].at[:, pl.ds(0, ci)], sis[buf]),
            pltpu.make_async_copy(
                rec_hbm.at[b, i, pl.ds(co, SZ[j])],
                recs[buf].at[pl.ds(0, SZ[j])], srs[buf]),
        )

    def out_copy(p, j, buf):
        b, i, co = rowcol(p, j)
        ci = 2 * SZ[j]
        return pltpu.make_async_copy(
            outs[buf].at[:, pl.ds(0, ci)],
            out_hbm.at[b, pl.ds(2 * i, 2), pl.ds(2 * co, ci)],
            sos[buf])

    def start_in(p, j, buf):
        for cp in in_copies(p, j, buf):
            cp.start()

    def wait_in(p, j, buf):
        for cp in in_copies(p, j, buf):
            cp.wait()

    def compute(j, buf):
        ib, rb, ob = ins[buf], recs[buf], outs[buf]
        nw = SZ[j]
        step = 32 // nw

        @plsc.parallel_loop(0, GRP, step=step, unroll=2)
        def grp_loop(g0):
            for jw in range(nw):
                for u in range(step):
                    sl = pl.ds(g0 * LANES + u * LANES, LANES)
                    a = ib[0, 2 * jw, sl]
                    bb = ib[0, 2 * jw + 1, sl]
                    cc = ib[1, 2 * jw, sl]
                    dd = ib[1, 2 * jw + 1, sl]
                    r = rb[jw, sl]
                    m = jnp.maximum(jnp.maximum(a, bb), jnp.maximum(cc, dd))
                    z = jnp.zeros((LANES,), jnp.float32)
                    c0 = a == m
                    c1 = bb == m
                    c2 = cc == m
                    t1 = jnp.where(c0, z, r)
                    t2 = jnp.where(c1, z, t1)
                    ob[0, 2 * jw, sl] = jnp.where(c0, r, z)
                    ob[0, 2 * jw + 1, sl] = jnp.where(c1, t1, z)
                    ob[1, 2 * jw, sl] = jnp.where(c2, t2, z)
                    ob[1, 2 * jw + 1, sl] = jnp.where(c2, z, t2)

    start_in(0, 0, 0)

    def main_loop(p, carry):
        for j in range(4):
            buf = j % 2
            if j < 3:
                start_in(p, j + 1, 1 - buf)
            else:
                @pl.when(p < PER_W - 1)
                def _():
                    start_in(p + 1, 0, 1 - buf)

            if j >= 2:
                out_copy(p, j - 2, buf).wait()
            else:
                @pl.when(p >= 1)
                def _():
                    out_copy(p - 1, j + 2, buf).wait()

            wait_in(p, j, buf)
            compute(j, buf)
            out_copy(p, j, buf).start()
        return carry

    lax.fori_loop(0, PER_W, main_loop, 0)
    out_copy(PER_W - 1, 2, 0).wait()
    out_copy(PER_W - 1, 3, 1).wait()


@jax.jit
def kernel(pool_input, recreated_output):
    run = pl.kernel(
        _unpool_body,
        out_type=jax.ShapeDtypeStruct((B, H, W, C), jnp.float32),
        scratch_types=[
            pltpu.VMEM((2, 2 * CMAX, C), jnp.float32),
            pltpu.VMEM((2, 2 * CMAX, C), jnp.float32),
            pltpu.VMEM((CMAX, C), jnp.float32),
            pltpu.VMEM((CMAX, C), jnp.float32),
            pltpu.VMEM((2, 2 * CMAX, C), jnp.float32),
            pltpu.VMEM((2, 2 * CMAX, C), jnp.float32),
            pltpu.SemaphoreType.DMA,
            pltpu.SemaphoreType.DMA,
            pltpu.SemaphoreType.DMA,
            pltpu.SemaphoreType.DMA,
            pltpu.SemaphoreType.DMA,
            pltpu.SemaphoreType.DMA,
        ],
        mesh=plsc.VectorSubcoreMesh(core_axis_name="c", subcore_axis_name="s"),
    )
    return run(pool_input, recreated_output)


# speedup vs baseline: 3.5236x; 1.0757x over previous
import jax
import jax.numpy as jnp
from jax import lax
from jax.experimental import pallas as pl
from jax.experimental.pallas import tpu as pltpu
from jax.experimental.pallas import tpu_sc as plsc

B, H, W, C = 8, 112, 112, 384
HO, WO = H // 2, W // 2
NW = 32
PAIRS = B * HO
PER_W = PAIRS // NW
CO = 8
CI = 2 * CO
KPP = WO // CO
NT = PER_W * KPP
LANES = 16
GRP = C // LANES
UNROLL = 8


def _unpool_body(pool_hbm, rec_hbm, out_hbm,
                 in0, in1, rec0, rec1, o0, o1,
                 si0, si1, sr0, sr1, so0, so1):
    wid = lax.axis_index("s") * 2 + lax.axis_index("c")
    ins, recs, outs = (in0, in1), (rec0, rec1), (o0, o1)
    sis, srs, sos = (si0, si1), (sr0, sr1), (so0, so1)

    def idx(t):
        pair = wid * PER_W + t // KPP
        k = t % KPP
        b = pair // HO
        i = pair - b * HO
        return b, i, k

    def in_copies(t, buf):
        b, i, k = idx(t)
        return (
            pltpu.make_async_copy(
                pool_hbm.at[b, pl.ds(2 * i, 2), pl.ds(CI * k, CI)],
                ins[buf], sis[buf]),
            pltpu.make_async_copy(
                rec_hbm.at[b, i, pl.ds(CO * k, CO)],
                recs[buf], srs[buf]),
        )

    def out_copy(t, buf):
        b, i, k = idx(t)
        return pltpu.make_async_copy(
            outs[buf],
            out_hbm.at[b, pl.ds(2 * i, 2), pl.ds(CI * k, CI)],
            sos[buf])

    def start_in(t, buf):
        c1, c2 = in_copies(t, buf)
        c1.start()
        c2.start()

    def wait_in(t, buf):
        c1, c2 = in_copies(t, buf)
        c1.wait()
        c2.wait()

    def compute(buf):
        ib, rb, ob = ins[buf], recs[buf], outs[buf]

        @plsc.parallel_loop(0, GRP, step=UNROLL, unroll=1)
        def grp_loop(g0):
            for jw in range(CO):
                for u in range(UNROLL):
                    sl = pl.ds(g0 * LANES + u * LANES, LANES)
                    a = ib[0, 2 * jw, sl]
                    bb = ib[0, 2 * jw + 1, sl]
                    cc = ib[1, 2 * jw, sl]
                    dd = ib[1, 2 * jw + 1, sl]
                    r = rb[jw, sl]
                    m = jnp.maximum(jnp.maximum(a, bb), jnp.maximum(cc, dd))
                    z = jnp.zeros((LANES,), jnp.float32)
                    c0 = a == m
                    c1 = bb == m
                    c2 = cc == m
                    t1 = jnp.where(c0, z, r)
                    t2 = jnp.where(c1, z, t1)
                    ob[0, 2 * jw, sl] = jnp.where(c0, r, z)
                    ob[0, 2 * jw + 1, sl] = jnp.where(c1, t1, z)
                    ob[1, 2 * jw, sl] = jnp.where(c2, t2, z)
                    ob[1, 2 * jw + 1, sl] = jnp.where(c2, z, t2)

    start_in(0, 0)

    def main_loop(tt, carry):
        for s in range(2):
            t = 2 * tt + s
            buf = s
            if s == 0:
                start_in(t + 1, 1)
            else:
                @pl.when(tt < NT // 2 - 1)
                def _():
                    start_in(t + 1, 0)

            @pl.when(tt >= 1)
            def _():
                out_copy(t - 2, buf).wait()

            wait_in(t, buf)
            compute(buf)
            out_copy(t, buf).start()
        return carry

    lax.fori_loop(0, NT // 2, main_loop, 0)
    out_copy(NT - 2, 0).wait()
    out_copy(NT - 1, 1).wait()


@jax.jit
def kernel(pool_input, recreated_output):
    run = pl.kernel(
        _unpool_body,
        out_type=jax.ShapeDtypeStruct((B, H, W, C), jnp.float32),
        scratch_types=[
            pltpu.VMEM((2, CI, C), jnp.float32),
            pltpu.VMEM((2, CI, C), jnp.float32),
            pltpu.VMEM((CO, C), jnp.float32),
            pltpu.VMEM((CO, C), jnp.float32),
            pltpu.VMEM((2, CI, C), jnp.float32),
            pltpu.VMEM((2, CI, C), jnp.float32),
            pltpu.SemaphoreType.DMA,
            pltpu.SemaphoreType.DMA,
            pltpu.SemaphoreType.DMA,
            pltpu.SemaphoreType.DMA,
            pltpu.SemaphoreType.DMA,
            pltpu.SemaphoreType.DMA,
        ],
        mesh=plsc.VectorSubcoreMesh(core_axis_name="c", subcore_axis_name="s"),
    )
    return run(pool_input, recreated_output)
